# Initial kernel scaffold; baseline (speedup 1.0000x reference)
import jax
import jax.numpy as jnp
from jax.experimental import pallas as pl


def kernel(inputs, pos_table):
    raise NotImplementedError("write your pallas kernel here")


# speedup vs baseline: 1.7567x; 1.7567x over previous
import jax
import jax.numpy as jnp
from jax.experimental import pallas as pl


_BS = 256


def _add_body(in_ref, pos_ref, out_ref):
    out_ref[...] = in_ref[...] + pos_ref[...][None]


def kernel(inputs, pos_table):
    batch, seq_len, dim = inputs.shape
    grid = (seq_len // _BS,)
    return pl.pallas_call(
        _add_body,
        grid=grid,
        in_specs=[
            pl.BlockSpec((batch, _BS, dim), lambda s: (0, s, 0)),
            pl.BlockSpec((_BS, dim), lambda s: (s, 0)),
        ],
        out_specs=pl.BlockSpec((batch, _BS, dim), lambda s: (0, s, 0)),
        out_shape=jax.ShapeDtypeStruct((batch, seq_len, dim), inputs.dtype),
    )(inputs, pos_table)


# speedup vs baseline: 1.8016x; 1.0256x over previous
import jax
import jax.numpy as jnp
from jax.experimental import pallas as pl


_BS = 512


def _add_body(in_ref, pos_ref, out_ref):
    out_ref[...] = in_ref[...] + pos_ref[...][None]


def kernel(inputs, pos_table):
    batch, seq_len, dim = inputs.shape
    grid = (seq_len // _BS,)
    return pl.pallas_call(
        _add_body,
        grid=grid,
        in_specs=[
            pl.BlockSpec((batch, _BS, dim), lambda s: (0, s, 0)),
            pl.BlockSpec((_BS, dim), lambda s: (s, 0)),
        ],
        out_specs=pl.BlockSpec((batch, _BS, dim), lambda s: (0, s, 0)),
        out_shape=jax.ShapeDtypeStruct((batch, seq_len, dim), inputs.dtype),
    )(inputs, pos_table)
